# baseline (speedup 1.0000x reference)
import functools

import jax
import jax.numpy as jnp
from jax import lax
from jax.experimental import pallas as pl
from jax.experimental.pallas import tpu as pltpu
from jax.experimental.pallas import tpu_sc as plsc

N_TOK = 8192
H = 768
H2 = H // 2
K = 1024
NC = 4
KS = NC * K
BLK = 256
FBLK = 1024

NCORE = 2
NSUB = 16
NW = NCORE * NSUB
TPW = N_TOK // NW
CH = 8
CHNC = CH * NC
NCHUNK = TPW // CH
DEPTH = 4
RPW = KS // NW
CR = 16

_BF = jnp.bfloat16
_DN = (((1,), (1,)), ((), ()))
_DN0 = (((1,), (0,)), ((), ()))
_ILV = plsc.PackFormat.INTERLEAVED


def _mm(a, b, dn=_DN0):
    return jax.lax.dot_general(a, b, dn, preferred_element_type=jnp.float32)


def _c2_body(cbs_ref, c2_ref):
    sq = cbs_ref[...]
    sq = sq * sq
    c2col = jnp.sum(sq, axis=1, keepdims=True)
    c2_ref[...] = jnp.broadcast_to(c2col.T, (8, KS))


CCH = 256


def _route_body(x_ref, cb16_ref, c2_ref, rW_ref, rb_ref,
                gidx_ref, w_ref):
    x = x_ref[...]
    xb = x.astype(_BF)
    x2 = jnp.sum(x * x, axis=1, keepdims=True)

    logits = _mm(xb, rW_ref[...].astype(_BF)) + rb_ref[...]
    m = jnp.max(logits, axis=1, keepdims=True)
    e = jnp.exp(logits - m)
    w_ref[...] = e / jnp.sum(e, axis=1, keepdims=True)

    iota = lax.broadcasted_iota(jnp.int32, (BLK, CCH), 1)
    cols = []
    for i in range(NC):
        cbi = cb16_ref[i * K:(i + 1) * K, :]
        s = _mm(xb, cbi, _DN)
        c2i = c2_ref[0:1, i * K:(i + 1) * K]

        def dchunk(c):
            sl = slice(c * CCH, (c + 1) * CCH)
            return (x2 - 2.0 * s[:, sl]) + c2i[:, sl]

        mi = jnp.min(dchunk(0), axis=1, keepdims=True)
        for c in range(1, K // CCH):
            mi = jnp.minimum(mi, jnp.min(dchunk(c), axis=1, keepdims=True))
        km = None
        for c in range(K // CCH):
            idc = jnp.where(dchunk(c) <= mi, iota, KS)
            kc = jnp.min(idc, axis=1, keepdims=True) + c * CCH
            km = kc if km is None else jnp.minimum(km, kc)
        cols.append(km + i * K)
    gidx_ref[...] = jnp.concatenate(cols, axis=1)


def _pack_body(cbs_hbm, cb32_hbm, inb, outp):
    cid = lax.axis_index("c")
    sid = lax.axis_index("s")
    wid = sid * NCORE + cid
    rbase = wid * RPW

    def chunk(cc, carry):
        rb = rbase + cc * CR
        pltpu.sync_copy(cbs_hbm.at[pl.ds(rb, CR)], inb)

        @plsc.parallel_loop(0, CR)
        def row(r):
            for g in range(H2 // 16):
                a = inb[r, pl.ds(16 * g, 16)]
                bseg = inb[r, pl.ds(H2 + 16 * g, 16)]
                p = plsc.pack(a, bseg, format=_ILV)
                outp[r, pl.ds(16 * g, 16)] = plsc.bitcast(p, jnp.float32)

        pltpu.sync_copy(outp, cb32_hbm.at[pl.ds(rb, CR)])
        return carry

    lax.fori_loop(0, RPW // CR, chunk, 0)


def _sc_body(cb32_hbm, gidx_hbm, w_hbm, out_hbm,
             idx_all, w_all, rows, outb, gsems, osems):
    cid = lax.axis_index("c")
    sid = lax.axis_index("s")
    wid = sid * NCORE + cid
    base = wid * TPW

    pltpu.sync_copy(gidx_hbm.at[pl.ds(base * NC, TPW * NC)], idx_all)
    pltpu.sync_copy(w_hbm.at[pl.ds(base * NC, TPW * NC)], w_all)

    def idxsl(ch):
        return idx_all.at[pl.ds(ch * CHNC, CHNC)]

    for b in range(DEPTH):
        pltpu.async_copy(cb32_hbm.at[idxsl(b)], rows.at[b], gsems[b])

    def grp(g, carry):
        for b in range(DEPTH):
            ch = DEPTH * g + b
            tb = base + ch * CH
            pltpu.make_async_copy(
                cb32_hbm.at[idxsl(ch)], rows.at[b], gsems[b]).wait()

            @pl.when(g > 0)
            def _():
                pltpu.make_async_copy(
                    outb.at[b], out_hbm.at[pl.ds(tb - DEPTH * CH, CH)],
                    osems[b]).wait()

            @plsc.parallel_loop(0, CH)
            def tok(t):
                zi = jnp.zeros((16,), jnp.int32)
                wbase = ch * CHNC + NC * t
                ws = [plsc.load_gather(w_all, [zi + (wbase + i)])
                      for i in range(NC)]
                r0 = NC * t
                for f in range(H2 // 16):
                    sl = pl.ds(f * 16, 16)
                    ab = [plsc.unpack(plsc.bitcast(rows[b, r0 + i, sl], _BF),
                                      format=_ILV,
                                      preferred_element_type=jnp.float32)
                          for i in range(NC)]
                    u = ws[0] * ab[0][0]
                    v = ws[0] * ab[0][1]
                    for i in range(1, NC):
                        u = u + ws[i] * ab[i][0]
                        v = v + ws[i] * ab[i][1]
                    packed = plsc.pack(u, v, format=_ILV)
                    outb[b, t, sl] = plsc.bitcast(packed, jnp.float32)

            pltpu.async_copy(outb.at[b], out_hbm.at[pl.ds(tb, CH)], osems[b])

            @pl.when(ch + DEPTH < NCHUNK)
            def _():
                pltpu.async_copy(cb32_hbm.at[idxsl(ch + DEPTH)], rows.at[b],
                                 gsems[b])
        return carry

    lax.fori_loop(0, NCHUNK // DEPTH, grp, 0)
    for b in range(DEPTH):
        pltpu.make_async_copy(
            outb.at[b],
            out_hbm.at[pl.ds(base + (NCHUNK - DEPTH + b) * CH, CH)],
            osems[b]).wait()


def _sc_gather_body(cb32_hbm, gidx_hbm, w_hbm, out_hbm,
                    idx_all, w_all, rows, outb,
                    g0, g1, g2, g3, o0, o1, o2, o3):
    _sc_body(cb32_hbm, gidx_hbm, w_hbm, out_hbm, idx_all, w_all,
             rows, outb, (g0, g1, g2, g3), (o0, o1, o2, o3))


def _final_body(comb_ref, iWlo_ref, iWhi_ref, ib_ref, out_ref):
    cu = lax.bitcast_convert_type(comb_ref[...], jnp.uint32)
    lo = lax.bitcast_convert_type(cu << jnp.uint32(16), jnp.float32)
    hi = lax.bitcast_convert_type(cu & jnp.uint32(0xFFFF0000), jnp.float32)
    out_ref[...] = (_mm(lo, iWlo_ref[...]) + _mm(hi, iWhi_ref[...])) \
        + ib_ref[...]


@jax.jit
def kernel(inputs, router_W, router_b, integ_W, integ_b, codebooks):
    flat = inputs.reshape(N_TOK, H)
    cbs = codebooks.reshape(KS, H)
    cb16 = cbs.astype(_BF)

    c2 = pl.pallas_call(
        _c2_body,
        grid=(1,),
        in_specs=[pl.BlockSpec((KS, H), lambda i: (0, 0))],
        out_specs=pl.BlockSpec((8, KS), lambda i: (0, 0)),
        out_shape=jax.ShapeDtypeStruct((8, KS), jnp.float32),
    )(cbs)

    gidx, w = pl.pallas_call(
        _route_body,
        grid=(N_TOK // BLK,),
        in_specs=[
            pl.BlockSpec((BLK, H), lambda i: (i, 0)),
            pl.BlockSpec((KS, H), lambda i: (0, 0)),
            pl.BlockSpec((8, KS), lambda i: (0, 0)),
            pl.BlockSpec((H, NC), lambda i: (0, 0)),
            pl.BlockSpec((1, NC), lambda i: (0, 0)),
        ],
        out_specs=[
            pl.BlockSpec((BLK, NC), lambda i: (i, 0)),
            pl.BlockSpec((BLK, NC), lambda i: (i, 0)),
        ],
        out_shape=[
            jax.ShapeDtypeStruct((N_TOK, NC), jnp.int32),
            jax.ShapeDtypeStruct((N_TOK, NC), jnp.float32),
        ],
    )(flat, cb16, c2, router_W, router_b.reshape(1, NC))

    mesh = plsc.VectorSubcoreMesh(core_axis_name="c", subcore_axis_name="s")

    sc_pack = functools.partial(
        pl.kernel,
        out_type=jax.ShapeDtypeStruct((KS, H2), jnp.float32),
        mesh=mesh,
        scratch_types=[
            pltpu.VMEM((CR, H), jnp.float32),
            pltpu.VMEM((CR, H2), jnp.float32),
        ],
        compiler_params=pltpu.CompilerParams(needs_layout_passes=False),
    )(_pack_body)
    cb32 = sc_pack(cbs)

    sc_gather = functools.partial(
        pl.kernel,
        out_type=jax.ShapeDtypeStruct((N_TOK, H2), jnp.float32),
        mesh=mesh,
        scratch_types=[
            pltpu.VMEM((TPW * NC,), jnp.int32),
            pltpu.VMEM((TPW * NC,), jnp.float32),
            pltpu.VMEM((DEPTH, CHNC, H2), jnp.float32),
            pltpu.VMEM((DEPTH, CH, H2), jnp.float32),
        ] + [pltpu.SemaphoreType.DMA] * (2 * DEPTH),
        compiler_params=pltpu.CompilerParams(needs_layout_passes=False),
    )(_sc_gather_body)
    combp = sc_gather(cb32, gidx.reshape(-1), w.reshape(-1))

    out = pl.pallas_call(
        _final_body,
        grid=(N_TOK // FBLK,),
        in_specs=[
            pl.BlockSpec((FBLK, H2), lambda i: (i, 0)),
            pl.BlockSpec((H2, H), lambda i: (0, 0)),
            pl.BlockSpec((H2, H), lambda i: (0, 0)),
            pl.BlockSpec((1, H), lambda i: (0, 0)),
        ],
        out_specs=pl.BlockSpec((FBLK, H), lambda i: (i, 0)),
        out_shape=jax.ShapeDtypeStruct((N_TOK, H), jnp.float32),
    )(combp, integ_W[:H2], integ_W[H2:], integ_b.reshape(1, H))

    return (out.reshape(inputs.shape),
            w.reshape(inputs.shape[0], inputs.shape[1], NC))


# speedup vs baseline: 1.0760x; 1.0760x over previous
import functools

import jax
import jax.numpy as jnp
from jax import lax
from jax.experimental import pallas as pl
from jax.experimental.pallas import tpu as pltpu
from jax.experimental.pallas import tpu_sc as plsc

N_TOK = 8192
H = 768
H2 = H // 2
K = 1024
NC = 4
KS = NC * K
BLK = 256
FBLK = 1024

NCORE = 2
NSUB = 16
NW = NCORE * NSUB
TPW = N_TOK // NW
CH = 8
CHNC = CH * NC
NCHUNK = TPW // CH
DEPTH = 4
RPW = KS // NW
CR = 16

_BF = jnp.bfloat16
_DN = (((1,), (1,)), ((), ()))
_DN0 = (((1,), (0,)), ((), ()))
_ILV = plsc.PackFormat.INTERLEAVED


def _mm(a, b, dn=_DN0):
    return jax.lax.dot_general(a, b, dn, preferred_element_type=jnp.float32)


def _c2_body(cbs_ref, c2_ref):
    sq = cbs_ref[...]
    sq = sq * sq
    c2col = jnp.sum(sq, axis=1, keepdims=True)
    c2_ref[...] = jnp.broadcast_to(c2col.T, (8, KS))


CCH = 256


def _route_body(x_ref, cb16_ref, c2_ref, rW_ref, rb_ref,
                gidx_ref, w_ref):
    x = x_ref[...]
    xb = x.astype(_BF)
    x2 = jnp.sum(x * x, axis=1, keepdims=True)

    logits = _mm(xb, rW_ref[...].astype(_BF)) + rb_ref[...]
    m = jnp.max(logits, axis=1, keepdims=True)
    e = jnp.exp(logits - m)
    w_ref[...] = e / jnp.sum(e, axis=1, keepdims=True)

    iota = lax.broadcasted_iota(jnp.int32, (BLK, K), 1)
    cols = []
    for i in range(NC):
        cbi = cb16_ref[i * K:(i + 1) * K, :]
        s = _mm(xb, cbi, _DN)
        d = (x2 - 2.0 * s) + c2_ref[0:1, i * K:(i + 1) * K]
        mi = jnp.min(d, axis=1, keepdims=True)
        ids = jnp.where(d <= mi, iota, K)
        kmin = jnp.min(ids, axis=1, keepdims=True)
        cols.append(kmin + i * K)
    gidx_ref[...] = jnp.concatenate(cols, axis=1)


def _pack_body(cbs_hbm, cb32_hbm, inb, outp):
    cid = lax.axis_index("c")
    sid = lax.axis_index("s")
    wid = sid * NCORE + cid
    rbase = wid * RPW

    def chunk(cc, carry):
        rb = rbase + cc * CR
        pltpu.sync_copy(cbs_hbm.at[pl.ds(rb, CR)], inb)

        @plsc.parallel_loop(0, CR)
        def row(r):
            for g in range(H2 // 16):
                a = inb[r, pl.ds(16 * g, 16)]
                bseg = inb[r, pl.ds(H2 + 16 * g, 16)]
                p = plsc.pack(a, bseg, format=_ILV)
                outp[r, pl.ds(16 * g, 16)] = plsc.bitcast(p, jnp.float32)

        pltpu.sync_copy(outp, cb32_hbm.at[pl.ds(rb, CR)])
        return carry

    lax.fori_loop(0, RPW // CR, chunk, 0)


def _sc_body(cb32_hbm, gidx_hbm, w_hbm, out_hbm,
             idx_all, w_all, rows, outb, gsems, osems):
    cid = lax.axis_index("c")
    sid = lax.axis_index("s")
    wid = sid * NCORE + cid
    base = wid * TPW

    pltpu.sync_copy(gidx_hbm.at[pl.ds(base * NC, TPW * NC)], idx_all)
    pltpu.sync_copy(w_hbm.at[pl.ds(base * NC, TPW * NC)], w_all)

    def idxsl(ch):
        return idx_all.at[pl.ds(ch * CHNC, CHNC)]

    for b in range(DEPTH):
        pltpu.async_copy(cb32_hbm.at[idxsl(b)], rows.at[b], gsems[b])

    def grp(g, carry):
        for b in range(DEPTH):
            ch = DEPTH * g + b
            tb = base + ch * CH
            pltpu.make_async_copy(
                cb32_hbm.at[idxsl(ch)], rows.at[b], gsems[b]).wait()

            @pl.when(g > 0)
            def _():
                pltpu.make_async_copy(
                    outb.at[b], out_hbm.at[pl.ds(tb - DEPTH * CH, CH)],
                    osems[b]).wait()

            @plsc.parallel_loop(0, CH)
            def tok(t):
                zi = jnp.zeros((16,), jnp.int32)
                wbase = ch * CHNC + NC * t
                ws = [plsc.load_gather(w_all, [zi + (wbase + i)])
                      for i in range(NC)]
                r0 = NC * t
                for f in range(H2 // 16):
                    sl = pl.ds(f * 16, 16)
                    ab = [plsc.unpack(plsc.bitcast(rows[b, r0 + i, sl], _BF),
                                      format=_ILV,
                                      preferred_element_type=jnp.float32)
                          for i in range(NC)]
                    u = ws[0] * ab[0][0]
                    v = ws[0] * ab[0][1]
                    for i in range(1, NC):
                        u = u + ws[i] * ab[i][0]
                        v = v + ws[i] * ab[i][1]
                    packed = plsc.pack(u, v, format=_ILV)
                    outb[b, t, sl] = plsc.bitcast(packed, jnp.float32)

            pltpu.async_copy(outb.at[b], out_hbm.at[pl.ds(tb, CH)], osems[b])

            @pl.when(ch + DEPTH < NCHUNK)
            def _():
                pltpu.async_copy(cb32_hbm.at[idxsl(ch + DEPTH)], rows.at[b],
                                 gsems[b])
        return carry

    lax.fori_loop(0, NCHUNK // DEPTH, grp, 0)
    for b in range(DEPTH):
        pltpu.make_async_copy(
            outb.at[b],
            out_hbm.at[pl.ds(base + (NCHUNK - DEPTH + b) * CH, CH)],
            osems[b]).wait()


def _sc_gather_body(cb32_hbm, gidx_hbm, w_hbm, out_hbm,
                    idx_all, w_all, rows, outb,
                    g0, g1, g2, g3, o0, o1, o2, o3):
    _sc_body(cb32_hbm, gidx_hbm, w_hbm, out_hbm, idx_all, w_all,
             rows, outb, (g0, g1, g2, g3), (o0, o1, o2, o3))


def _final_body(comb_ref, iWlo_ref, iWhi_ref, ib_ref, out_ref):
    cu = lax.bitcast_convert_type(comb_ref[...], jnp.uint32)
    lo = lax.bitcast_convert_type(cu << jnp.uint32(16), jnp.float32)
    hi = lax.bitcast_convert_type(cu & jnp.uint32(0xFFFF0000), jnp.float32)
    out_ref[...] = (_mm(lo, iWlo_ref[...]) + _mm(hi, iWhi_ref[...])) \
        + ib_ref[...]


@jax.jit
def kernel(inputs, router_W, router_b, integ_W, integ_b, codebooks):
    flat = inputs.reshape(N_TOK, H)
    cbs = codebooks.reshape(KS, H)
    cb16 = cbs.astype(_BF)

    c2 = pl.pallas_call(
        _c2_body,
        grid=(1,),
        in_specs=[pl.BlockSpec((KS, H), lambda i: (0, 0))],
        out_specs=pl.BlockSpec((8, KS), lambda i: (0, 0)),
        out_shape=jax.ShapeDtypeStruct((8, KS), jnp.float32),
    )(cbs)

    gidx, w = pl.pallas_call(
        _route_body,
        grid=(N_TOK // BLK,),
        in_specs=[
            pl.BlockSpec((BLK, H), lambda i: (i, 0)),
            pl.BlockSpec((KS, H), lambda i: (0, 0)),
            pl.BlockSpec((8, KS), lambda i: (0, 0)),
            pl.BlockSpec((H, NC), lambda i: (0, 0)),
            pl.BlockSpec((1, NC), lambda i: (0, 0)),
        ],
        out_specs=[
            pl.BlockSpec((BLK, NC), lambda i: (i, 0)),
            pl.BlockSpec((BLK, NC), lambda i: (i, 0)),
        ],
        out_shape=[
            jax.ShapeDtypeStruct((N_TOK, NC), jnp.int32),
            jax.ShapeDtypeStruct((N_TOK, NC), jnp.float32),
        ],
    )(flat, cb16, c2, router_W, router_b.reshape(1, NC))

    mesh = plsc.VectorSubcoreMesh(core_axis_name="c", subcore_axis_name="s")

    sc_pack = functools.partial(
        pl.kernel,
        out_type=jax.ShapeDtypeStruct((KS, H2), jnp.float32),
        mesh=mesh,
        scratch_types=[
            pltpu.VMEM((CR, H), jnp.float32),
            pltpu.VMEM((CR, H2), jnp.float32),
        ],
        compiler_params=pltpu.CompilerParams(needs_layout_passes=False),
    )(_pack_body)
    cb32 = sc_pack(cbs)

    sc_gather = functools.partial(
        pl.kernel,
        out_type=jax.ShapeDtypeStruct((N_TOK, H2), jnp.float32),
        mesh=mesh,
        scratch_types=[
            pltpu.VMEM((TPW * NC,), jnp.int32),
            pltpu.VMEM((TPW * NC,), jnp.float32),
            pltpu.VMEM((DEPTH, CHNC, H2), jnp.float32),
            pltpu.VMEM((DEPTH, CH, H2), jnp.float32),
        ] + [pltpu.SemaphoreType.DMA] * (2 * DEPTH),
        compiler_params=pltpu.CompilerParams(needs_layout_passes=False),
    )(_sc_gather_body)
    combp = sc_gather(cb32, gidx.reshape(-1), w.reshape(-1))

    out = pl.pallas_call(
        _final_body,
        grid=(N_TOK // FBLK,),
        in_specs=[
            pl.BlockSpec((FBLK, H2), lambda i: (i, 0)),
            pl.BlockSpec((H2, H), lambda i: (0, 0)),
            pl.BlockSpec((H2, H), lambda i: (0, 0)),
            pl.BlockSpec((1, H), lambda i: (0, 0)),
        ],
        out_specs=pl.BlockSpec((FBLK, H), lambda i: (i, 0)),
        out_shape=jax.ShapeDtypeStruct((N_TOK, H), jnp.float32),
    )(combp, integ_W[:H2], integ_W[H2:], integ_b.reshape(1, H))

    return (out.reshape(inputs.shape),
            w.reshape(inputs.shape[0], inputs.shape[1], NC))


# speedup vs baseline: 1.0968x; 1.0194x over previous
import functools

import jax
import jax.numpy as jnp
from jax import lax
from jax.experimental import pallas as pl
from jax.experimental.pallas import tpu as pltpu
from jax.experimental.pallas import tpu_sc as plsc

N_TOK = 8192
H = 768
H2 = H // 2
K = 1024
NC = 4
KS = NC * K
BLK = 256
FBLK = 1024

NCORE = 2
NSUB = 16
NW = NCORE * NSUB
TPW = N_TOK // NW
CH = 16
CHNC = CH * NC
NCHUNK = TPW // CH
DEPTH = 2
RPW = KS // NW
CR = 16

_BF = jnp.bfloat16
_DN = (((1,), (1,)), ((), ()))
_DN0 = (((1,), (0,)), ((), ()))
_ILV = plsc.PackFormat.INTERLEAVED


def _mm(a, b, dn=_DN0):
    return jax.lax.dot_general(a, b, dn, preferred_element_type=jnp.float32)


def _c2_body(cbs_ref, c2_ref):
    sq = cbs_ref[...]
    sq = sq * sq
    c2col = jnp.sum(sq, axis=1, keepdims=True)
    c2_ref[...] = jnp.broadcast_to(c2col.T, (8, KS))


CCH = 256


def _route_body(x_ref, cb16_ref, c2_ref, rW_ref, rb_ref,
                gidx_ref, w_ref):
    x = x_ref[...]
    xb = x.astype(_BF)
    x2 = jnp.sum(x * x, axis=1, keepdims=True)

    logits = _mm(xb, rW_ref[...].astype(_BF)) + rb_ref[...]
    m = jnp.max(logits, axis=1, keepdims=True)
    e = jnp.exp(logits - m)
    w_ref[...] = e / jnp.sum(e, axis=1, keepdims=True)

    iota = lax.broadcasted_iota(jnp.int32, (BLK, K), 1)
    cols = []
    for i in range(NC):
        cbi = cb16_ref[i * K:(i + 1) * K, :]
        s = _mm(xb, cbi, _DN)
        d = (x2 - 2.0 * s) + c2_ref[0:1, i * K:(i + 1) * K]
        mi = jnp.min(d, axis=1, keepdims=True)
        ids = jnp.where(d <= mi, iota, K)
        kmin = jnp.min(ids, axis=1, keepdims=True)
        cols.append(kmin + i * K)
    gidx_ref[...] = jnp.concatenate(cols, axis=1)


def _pack_body(cbs_hbm, cb32_hbm, inb, outp):
    cid = lax.axis_index("c")
    sid = lax.axis_index("s")
    wid = sid * NCORE + cid
    rbase = wid * RPW

    def chunk(cc, carry):
        rb = rbase + cc * CR
        pltpu.sync_copy(cbs_hbm.at[pl.ds(rb, CR)], inb)

        @plsc.parallel_loop(0, CR)
        def row(r):
            for g in range(H2 // 16):
                a = inb[r, pl.ds(16 * g, 16)]
                bseg = inb[r, pl.ds(H2 + 16 * g, 16)]
                p = plsc.pack(a, bseg, format=_ILV)
                outp[r, pl.ds(16 * g, 16)] = plsc.bitcast(p, jnp.float32)

        pltpu.sync_copy(outp, cb32_hbm.at[pl.ds(rb, CR)])
        return carry

    lax.fori_loop(0, RPW // CR, chunk, 0)


def _sc_body(cb32_hbm, gidx_hbm, w_hbm, out_hbm,
             idx_all, w_all, rows, outb, gsems, osems):
    cid = lax.axis_index("c")
    sid = lax.axis_index("s")
    wid = sid * NCORE + cid
    base = wid * TPW

    pltpu.sync_copy(gidx_hbm.at[pl.ds(base * NC, TPW * NC)], idx_all)
    pltpu.sync_copy(w_hbm.at[pl.ds(base * NC, TPW * NC)], w_all)

    def idxsl(ch):
        return idx_all.at[pl.ds(ch * CHNC, CHNC)]

    for b in range(DEPTH):
        pltpu.async_copy(cb32_hbm.at[idxsl(b)], rows.at[b], gsems[b])

    def grp(g, carry):
        for b in range(DEPTH):
            ch = DEPTH * g + b
            tb = base + ch * CH
            pltpu.make_async_copy(
                cb32_hbm.at[idxsl(ch)], rows.at[b], gsems[b]).wait()

            @pl.when(g > 0)
            def _():
                pltpu.make_async_copy(
                    outb.at[b], out_hbm.at[pl.ds(tb - DEPTH * CH, CH)],
                    osems[b]).wait()

            @plsc.parallel_loop(0, CH)
            def tok(t):
                zi = jnp.zeros((16,), jnp.int32)
                wbase = ch * CHNC + NC * t
                ws = [plsc.load_gather(w_all, [zi + (wbase + i)])
                      for i in range(NC)]
                r0 = NC * t
                for f in range(H2 // 16):
                    sl = pl.ds(f * 16, 16)
                    ab = [plsc.unpack(plsc.bitcast(rows[b, r0 + i, sl], _BF),
                                      format=_ILV,
                                      preferred_element_type=jnp.float32)
                          for i in range(NC)]
                    u = ws[0] * ab[0][0]
                    v = ws[0] * ab[0][1]
                    for i in range(1, NC):
                        u = u + ws[i] * ab[i][0]
                        v = v + ws[i] * ab[i][1]
                    packed = plsc.pack(u, v, format=_ILV)
                    outb[b, t, sl] = plsc.bitcast(packed, jnp.float32)

            pltpu.async_copy(outb.at[b], out_hbm.at[pl.ds(tb, CH)], osems[b])

            @pl.when(ch + DEPTH < NCHUNK)
            def _():
                pltpu.async_copy(cb32_hbm.at[idxsl(ch + DEPTH)], rows.at[b],
                                 gsems[b])
        return carry

    lax.fori_loop(0, NCHUNK // DEPTH, grp, 0)
    for b in range(DEPTH):
        pltpu.make_async_copy(
            outb.at[b],
            out_hbm.at[pl.ds(base + (NCHUNK - DEPTH + b) * CH, CH)],
            osems[b]).wait()


def _sc_gather_body(cb32_hbm, gidx_hbm, w_hbm, out_hbm,
                    idx_all, w_all, rows, outb,
                    g0, g1, o0, o1):
    _sc_body(cb32_hbm, gidx_hbm, w_hbm, out_hbm, idx_all, w_all,
             rows, outb, (g0, g1), (o0, o1))


def _final_body(comb_ref, iWlo_ref, iWhi_ref, ib_ref, out_ref):
    cu = lax.bitcast_convert_type(comb_ref[...], jnp.uint32)
    lo = lax.bitcast_convert_type(cu << jnp.uint32(16), jnp.float32)
    hi = lax.bitcast_convert_type(cu & jnp.uint32(0xFFFF0000), jnp.float32)
    out_ref[...] = (_mm(lo, iWlo_ref[...]) + _mm(hi, iWhi_ref[...])) \
        + ib_ref[...]


@jax.jit
def kernel(inputs, router_W, router_b, integ_W, integ_b, codebooks):
    flat = inputs.reshape(N_TOK, H)
    cbs = codebooks.reshape(KS, H)
    cb16 = cbs.astype(_BF)

    c2 = pl.pallas_call(
        _c2_body,
        grid=(1,),
        in_specs=[pl.BlockSpec((KS, H), lambda i: (0, 0))],
        out_specs=pl.BlockSpec((8, KS), lambda i: (0, 0)),
        out_shape=jax.ShapeDtypeStruct((8, KS), jnp.float32),
    )(cbs)

    gidx, w = pl.pallas_call(
        _route_body,
        grid=(N_TOK // BLK,),
        in_specs=[
            pl.BlockSpec((BLK, H), lambda i: (i, 0)),
            pl.BlockSpec((KS, H), lambda i: (0, 0)),
            pl.BlockSpec((8, KS), lambda i: (0, 0)),
            pl.BlockSpec((H, NC), lambda i: (0, 0)),
            pl.BlockSpec((1, NC), lambda i: (0, 0)),
        ],
        out_specs=[
            pl.BlockSpec((BLK, NC), lambda i: (i, 0)),
            pl.BlockSpec((BLK, NC), lambda i: (i, 0)),
        ],
        out_shape=[
            jax.ShapeDtypeStruct((N_TOK, NC), jnp.int32),
            jax.ShapeDtypeStruct((N_TOK, NC), jnp.float32),
        ],
    )(flat, cb16, c2, router_W, router_b.reshape(1, NC))

    mesh = plsc.VectorSubcoreMesh(core_axis_name="c", subcore_axis_name="s")

    sc_pack = functools.partial(
        pl.kernel,
        out_type=jax.ShapeDtypeStruct((KS, H2), jnp.float32),
        mesh=mesh,
        scratch_types=[
            pltpu.VMEM((CR, H), jnp.float32),
            pltpu.VMEM((CR, H2), jnp.float32),
        ],
        compiler_params=pltpu.CompilerParams(needs_layout_passes=False),
    )(_pack_body)
    cb32 = sc_pack(cbs)

    sc_gather = functools.partial(
        pl.kernel,
        out_type=jax.ShapeDtypeStruct((N_TOK, H2), jnp.float32),
        mesh=mesh,
        scratch_types=[
            pltpu.VMEM((TPW * NC,), jnp.int32),
            pltpu.VMEM((TPW * NC,), jnp.float32),
            pltpu.VMEM((DEPTH, CHNC, H2), jnp.float32),
            pltpu.VMEM((DEPTH, CH, H2), jnp.float32),
        ] + [pltpu.SemaphoreType.DMA] * (2 * DEPTH),
        compiler_params=pltpu.CompilerParams(needs_layout_passes=False),
    )(_sc_gather_body)
    combp = sc_gather(cb32, gidx.reshape(-1), w.reshape(-1))

    out = pl.pallas_call(
        _final_body,
        grid=(N_TOK // FBLK,),
        in_specs=[
            pl.BlockSpec((FBLK, H2), lambda i: (i, 0)),
            pl.BlockSpec((H2, H), lambda i: (0, 0)),
            pl.BlockSpec((H2, H), lambda i: (0, 0)),
            pl.BlockSpec((1, H), lambda i: (0, 0)),
        ],
        out_specs=pl.BlockSpec((FBLK, H), lambda i: (i, 0)),
        out_shape=jax.ShapeDtypeStruct((N_TOK, H), jnp.float32),
    )(combp, integ_W[:H2], integ_W[H2:], integ_b.reshape(1, H))

    return (out.reshape(inputs.shape),
            w.reshape(inputs.shape[0], inputs.shape[1], NC))
